# Initial kernel scaffold; baseline (speedup 1.0000x reference)
import jax
import jax.numpy as jnp
from jax.experimental import pallas as pl


def kernel(x_client, x_sku, edge_index_cs, edge_index_sc, params):
    raise NotImplementedError("write your pallas kernel here")


# speedup vs baseline: 1.0038x; 1.0038x over previous
import jax
import jax.numpy as jnp
from jax.experimental import pallas as pl

N_CLIENT = 10000
N_SKU = 10000
C = 128


def _linear_norm_kernel(x_ref, w_ref, b_ref, o_ref):
    x = x_ref[...]
    n = jnp.sqrt(jnp.sum(x * x, axis=-1, keepdims=True))
    xn = x / jnp.maximum(n, 1e-12)
    o_ref[...] = jnp.dot(xn, w_ref[...], preferred_element_type=jnp.float32) + b_ref[...]


def _lin_init(x, w, b):
    n = x.shape[0]
    blk = 1000
    return pl.pallas_call(
        _linear_norm_kernel,
        grid=(n // blk,),
        in_specs=[
            pl.BlockSpec((blk, x.shape[1]), lambda i: (i, 0)),
            pl.BlockSpec((x.shape[1], w.shape[1]), lambda i: (0, 0)),
            pl.BlockSpec((1, w.shape[1]), lambda i: (0, 0)),
        ],
        out_specs=pl.BlockSpec((blk, w.shape[1]), lambda i: (i, 0)),
        out_shape=jax.ShapeDtypeStruct((n, w.shape[1]), jnp.float32),
    )(x, w, b.reshape(1, -1))


def _l2norm(x):
    n = jnp.sqrt(jnp.sum(x * x, axis=-1, keepdims=True))
    return x / jnp.maximum(n, 1e-12)


def _layernorm(x, gamma, beta):
    m = jnp.mean(x, axis=-1, keepdims=True)
    v = jnp.var(x, axis=-1, keepdims=True)
    return (x - m) / jnp.sqrt(v + 1e-5) * gamma + beta


def _mean_aggr(x_src, src, dst, n_dst):
    msg = jnp.take(x_src, src, axis=0)
    s = jax.ops.segment_sum(msg, dst, num_segments=n_dst)
    cnt = jax.ops.segment_sum(jnp.ones((src.shape[0], 1), jnp.float32), dst, num_segments=n_dst)
    return s / jnp.maximum(cnt, 1.0)


def _graph_conv(x_src, x_dst, src, dst, p, n_dst):
    m = _mean_aggr(x_src, src, dst, n_dst)
    return m @ p['W_rel'] + p['b_rel'] + x_dst @ p['W_root']


def _head(x, p):
    (w1, b1), (w2, b2) = p
    h = jax.nn.relu(x @ w1 + b1)
    return jax.nn.sigmoid(h @ w2 + b2)


def kernel(x_client, x_sku, edge_index_cs, edge_index_sc, params):
    wic, bic = params['lin_init']['client']
    wis, bis = params['lin_init']['sku']
    xc = _lin_init(x_client, wic, bic)
    xs = _lin_init(x_sku, wis, bis)
    for conv, norm in zip(params['convs'], params['norms']):
        out_sku = _graph_conv(xc, xs, edge_index_cs[0], edge_index_cs[1], conv['cs'], N_SKU)
        out_client = _graph_conv(xs, xc, edge_index_sc[0], edge_index_sc[1], conv['sc'], N_CLIENT)
        gc, bc = norm['client']
        gs, bs = norm['sku']
        xc = jax.nn.relu(_layernorm(out_client, gc, bc))
        xs = jax.nn.relu(_layernorm(out_sku, gs, bs))
    user_emb = _l2norm(xc)
    churn = _head(user_emb, params['heads']['churn'])
    cat = _head(user_emb, params['heads']['cat'])
    sku = _head(user_emb, params['heads']['sku'])
    return (churn, cat, sku, user_emb)


# speedup vs baseline: 3.5519x; 3.5385x over previous
import functools

import jax
import jax.numpy as jnp
from jax import lax
from jax.experimental import pallas as pl
from jax.experimental.pallas import tpu as pltpu, tpu_sc as plsc

N_CLIENT = 10000
N_SKU = 10000
E = 320000
C = 128

NUM_TILES = 16
CHUNK = 128
NCH = -(-E // (NUM_TILES * CHUNK))
EPT = NCH * CHUNK
TOT = NUM_TILES * EPT
ACC_ROWS = 10240
DUMMY = 10000
ZROWS = 64
RPT = ACC_ROWS // NUM_TILES


HC = C // 2


def _aggr_body(x_lo, x_hi, src_idx, dst_idx, ones_hbm, z64_hbm, z16_hbm,
               sum_out, cnt_out,
               idx_s_v, idx_d_v, rows_v, ones_v, z64_v, z16_v, acc, cntacc,
               sem):
    c = lax.axis_index("c")
    s = lax.axis_index("s")

    pltpu.sync_copy(src_idx.at[c, s], idx_s_v)
    pltpu.sync_copy(dst_idx.at[c, s], idx_d_v)
    pltpu.sync_copy(ones_hbm, ones_v)
    pltpu.sync_copy(z64_hbm, z64_v)
    pltpu.sync_copy(z16_hbm, z16_v)

    base = s * RPT
    for p, table in enumerate((x_lo, x_hi)):
        for i in range(RPT // ZROWS):
            pltpu.sync_copy(z64_v, acc.at[pl.ds(base + i * ZROWS, ZROWS)])
            if p == 0:
                pltpu.sync_copy(z16_v,
                                cntacc.at[pl.ds(base + i * ZROWS, ZROWS)])
        plsc.subcore_barrier()

        def chunk(j, carry, table=table, p=p):
            pltpu.async_copy(table.at[idx_s_v.at[j]], rows_v, sem).wait()
            pltpu.sync_copy(rows_v, acc.at[idx_d_v.at[j]], add=True)
            if p == 0:
                pltpu.sync_copy(ones_v, cntacc.at[idx_d_v.at[j]], add=True)
            return carry

        lax.fori_loop(0, NCH, chunk, 0)
        plsc.subcore_barrier()

        pltpu.sync_copy(acc.at[pl.ds(base, RPT)],
                        sum_out.at[p, c, pl.ds(base, RPT)])
        if p == 0:
            pltpu.sync_copy(cntacc.at[pl.ds(base, RPT)],
                            cnt_out.at[c, pl.ds(base, RPT)])


def _make_aggr():
    out_type = [
        jax.ShapeDtypeStruct((2, 2, ACC_ROWS, HC), jnp.float32),
        jax.ShapeDtypeStruct((2, ACC_ROWS, 16), jnp.float32),
    ]
    mesh = plsc.VectorSubcoreMesh(core_axis_name="c", subcore_axis_name="s")
    return pl.kernel(
        _aggr_body,
        out_type=out_type,
        mesh=mesh,
        scratch_types=[
            pltpu.VMEM((NCH, CHUNK), jnp.int32),
            pltpu.VMEM((NCH, CHUNK), jnp.int32),
            pltpu.VMEM((CHUNK, HC), jnp.float32),
            pltpu.VMEM((CHUNK, 16), jnp.float32),
            pltpu.VMEM((ZROWS, HC), jnp.float32),
            pltpu.VMEM((ZROWS, 16), jnp.float32),
            pltpu.VMEM_SHARED((ACC_ROWS, HC), jnp.float32),
            pltpu.VMEM_SHARED((ACC_ROWS, 16), jnp.float32),
            pltpu.SemaphoreType.DMA,
        ],
        compiler_params=pltpu.CompilerParams(use_tc_tiling_on_sc=False),
    )


_aggr_cnt = _make_aggr()


def _prep_edges(edge_index_cs, edge_index_sc):
    def one(e_idx, src_off):
        src = jnp.zeros((TOT,), jnp.int32).at[:E].set(e_idx[0]) + src_off
        dst = jnp.full((TOT,), DUMMY, jnp.int32).at[:E].set(e_idx[1])
        return (src.reshape(NUM_TILES, NCH, CHUNK),
                dst.reshape(NUM_TILES, NCH, CHUNK))

    s_cs, d_cs = one(edge_index_cs, 0)
    s_sc, d_sc = one(edge_index_sc, N_CLIENT)
    return jnp.stack([s_cs, s_sc]), jnp.stack([d_cs, d_sc])


def _init_body(x_ref, w_ref, b_ref, o_ref):
    x = x_ref[...]
    n = jnp.sqrt(jnp.sum(x * x, axis=-1, keepdims=True))
    xn = x / jnp.maximum(n, 1e-12)
    o_ref[...] = jnp.dot(xn, w_ref[...],
                         preferred_element_type=jnp.float32) + b_ref[...]


def _tc_init(x, w, b):
    n, blk = x.shape[0], 2000
    return pl.pallas_call(
        _init_body,
        grid=(n // blk,),
        in_specs=[
            pl.BlockSpec((blk, x.shape[1]), lambda i: (i, 0)),
            pl.BlockSpec((x.shape[1], w.shape[1]), lambda i: (0, 0)),
            pl.BlockSpec((1, w.shape[1]), lambda i: (0, 0)),
        ],
        out_specs=pl.BlockSpec((blk, w.shape[1]), lambda i: (i, 0)),
        out_shape=jax.ShapeDtypeStruct((n, w.shape[1]), jnp.float32),
    )(x, w, b.reshape(1, -1))


def _post_body(s_ref, cnt_ref, xd_ref, wr_ref, br_ref, wo_ref, g_ref, be_ref,
               o_ref):
    m = s_ref[...] / jnp.maximum(cnt_ref[...], 1.0)
    h = (jnp.dot(m, wr_ref[...], preferred_element_type=jnp.float32)
         + br_ref[...]
         + jnp.dot(xd_ref[...], wo_ref[...], preferred_element_type=jnp.float32))
    mu = jnp.mean(h, axis=-1, keepdims=True)
    v = jnp.mean((h - mu) ** 2, axis=-1, keepdims=True)
    hn = (h - mu) / jnp.sqrt(v + 1e-5) * g_ref[...] + be_ref[...]
    o_ref[...] = jnp.maximum(hn, 0.0)


def _tc_post(s_sum, cnt, x_dst, conv_p, norm_p):
    n, blk = x_dst.shape[0], 2000
    gamma, beta = norm_p
    return pl.pallas_call(
        _post_body,
        grid=(n // blk,),
        in_specs=[
            pl.BlockSpec((blk, C), lambda i: (i, 0)),
            pl.BlockSpec((blk, 1), lambda i: (i, 0)),
            pl.BlockSpec((blk, C), lambda i: (i, 0)),
            pl.BlockSpec((C, C), lambda i: (0, 0)),
            pl.BlockSpec((1, C), lambda i: (0, 0)),
            pl.BlockSpec((C, C), lambda i: (0, 0)),
            pl.BlockSpec((1, C), lambda i: (0, 0)),
            pl.BlockSpec((1, C), lambda i: (0, 0)),
        ],
        out_specs=pl.BlockSpec((blk, C), lambda i: (i, 0)),
        out_shape=jax.ShapeDtypeStruct((n, C), jnp.float32),
    )(s_sum, cnt, x_dst, conv_p['W_rel'], conv_p['b_rel'].reshape(1, -1),
      conv_p['W_root'], gamma.reshape(1, -1), beta.reshape(1, -1))


def _heads_body(x_ref, w1c_ref, b1c_ref, w2c_ref, b2c_ref, w1a_ref, b1a_ref,
                w2a_ref, b2a_ref, w1s_ref, b1s_ref, w2s_ref, b2s_ref,
                churn_ref, cat_ref, sku_ref, ue_ref):
    x = x_ref[...]
    n = jnp.sqrt(jnp.sum(x * x, axis=-1, keepdims=True))
    ue = x / jnp.maximum(n, 1e-12)
    ue_ref[...] = ue

    def head(w1, b1, w2, b2):
        h = jnp.maximum(jnp.dot(ue, w1, preferred_element_type=jnp.float32)
                        + b1, 0.0)
        return jax.nn.sigmoid(jnp.dot(h, w2, preferred_element_type=jnp.float32)
                              + b2)

    churn_ref[...] = head(w1c_ref[...], b1c_ref[...], w2c_ref[...], b2c_ref[...])
    cat_ref[...] = head(w1a_ref[...], b1a_ref[...], w2a_ref[...], b2a_ref[...])
    sku_ref[...] = head(w1s_ref[...], b1s_ref[...], w2s_ref[...], b2s_ref[...])


def _tc_heads(x, heads):
    n, blk = x.shape[0], 2000
    (w1c, b1c), (w2c, b2c) = heads['churn']
    (w1a, b1a), (w2a, b2a) = heads['cat']
    (w1s, b1s), (w2s, b2s) = heads['sku']
    n_cat, n_sku = w2a.shape[1], w2s.shape[1]

    def full(shape):
        return pl.BlockSpec(shape, lambda i: tuple(0 for _ in shape))

    return pl.pallas_call(
        _heads_body,
        grid=(n // blk,),
        in_specs=[
            pl.BlockSpec((blk, C), lambda i: (i, 0)),
            full((C, 128)), full((1, 128)), full((128, 1)), full((1, 1)),
            full((C, 128)), full((1, 128)), full((128, n_cat)), full((1, n_cat)),
            full((C, 128)), full((1, 128)), full((128, n_sku)), full((1, n_sku)),
        ],
        out_specs=[
            pl.BlockSpec((blk, 1), lambda i: (i, 0)),
            pl.BlockSpec((blk, n_cat), lambda i: (i, 0)),
            pl.BlockSpec((blk, n_sku), lambda i: (i, 0)),
            pl.BlockSpec((blk, C), lambda i: (i, 0)),
        ],
        out_shape=[
            jax.ShapeDtypeStruct((n, 1), jnp.float32),
            jax.ShapeDtypeStruct((n, n_cat), jnp.float32),
            jax.ShapeDtypeStruct((n, n_sku), jnp.float32),
            jax.ShapeDtypeStruct((n, C), jnp.float32),
        ],
    )(x, w1c, b1c.reshape(1, -1), w2c, b2c.reshape(1, -1),
      w1a, b1a.reshape(1, -1), w2a, b2a.reshape(1, -1),
      w1s, b1s.reshape(1, -1), w2s, b2s.reshape(1, -1))


def kernel(x_client, x_sku, edge_index_cs, edge_index_sc, params):
    src_idx, dst_idx = _prep_edges(edge_index_cs, edge_index_sc)
    ones = jnp.ones((CHUNK, 16), jnp.float32)
    z128 = jnp.zeros((ZROWS, HC), jnp.float32)
    z16 = jnp.zeros((ZROWS, 16), jnp.float32)

    wic, bic = params['lin_init']['client']
    wis, bis = params['lin_init']['sku']
    xc = _tc_init(x_client, wic, bic)
    xs = _tc_init(x_sku, wis, bis)

    convs = jax.tree.map(lambda *a: jnp.stack(a), *params['convs'])
    norms = jax.tree.map(lambda *a: jnp.stack(a), *params['norms'])

    def layer_step(carry, layer_p):
        xc, xs = carry
        conv, norm = layer_p
        x_cat = jnp.concatenate([xc, xs], axis=0)
        sums, cnts = _aggr_cnt(x_cat[:, :HC], x_cat[:, HC:],
                               src_idx, dst_idx, ones, z128, z16)
        sums = jnp.concatenate([sums[0], sums[1]], axis=-1)
        cnt_s = cnts[0, :N_SKU, :1]
        cnt_c = cnts[1, :N_CLIENT, :1]
        xs_new = _tc_post(sums[0, :N_SKU], cnt_s, xs, conv['cs'], norm['sku'])
        xc_new = _tc_post(sums[1, :N_CLIENT], cnt_c, xc, conv['sc'],
                          norm['client'])
        return (xc_new, xs_new), None

    (xc, xs), _ = lax.scan(layer_step, (xc, xs), (convs, norms))

    churn, cat, sku, ue = _tc_heads(xc, params['heads'])
    return (churn, cat, sku, ue)
